# baseline (speedup 1.0000x reference)
import functools

import jax
import jax.numpy as jnp
from jax import lax
from jax.experimental import pallas as pl
from jax.experimental.pallas import tpu as pltpu
from jax.experimental.pallas import tpu_sc as plsc

NC = 2
NS = 16
NW = NC * NS
L = 16
CH = 128


def _make_sc_deg(NP, CPT):
    mesh = plsc.VectorSubcoreMesh(core_axis_name="c", subcore_axis_name="s")

    @functools.partial(
        pl.kernel,
        out_type=jax.ShapeDtypeStruct((NW * NP,), jnp.float32),
        mesh=mesh,
        scratch_types=[
            pltpu.VMEM((CPT * CH,), jnp.int32),
            pltpu.VMEM((NP,), jnp.float32),
        ],
        compiler_params=pltpu.CompilerParams(needs_layout_passes=False),
    )
    def deg_kernel(src_hbm, degp_hbm, idx_v, deg_v):
        c = lax.axis_index("c")
        s = lax.axis_index("s")
        wid = s * NC + c

        def zero_body(i, _):
            deg_v[pl.ds(i * L, L)] = jnp.zeros((L,), jnp.float32)
            return 0

        lax.fori_loop(0, NP // L, zero_body, 0)

        pltpu.sync_copy(src_hbm.at[pl.ds(wid * CPT * CH, CPT * CH)], idx_v)

        ones = jnp.ones((L,), jnp.float32)

        def body(j, _):
            for i in range(CH // L):
                iv = idx_v[pl.ds(j * CH + i * L, L)]
                plsc.addupdate_scatter(deg_v, [iv], ones)
            return 0

        lax.fori_loop(0, CPT, body, 0)
        pltpu.sync_copy(deg_v, degp_hbm.at[pl.ds(wid * NP, NP)])

    return deg_kernel


def _make_sc_segsum(NP, CPT):
    mesh = plsc.VectorSubcoreMesh(core_axis_name="c", subcore_axis_name="s")
    RPT = NP // NS
    G = 16
    NG = CPT // G
    NSLOT = 2 * G
    NACC = 4

    scratch = [pltpu.VMEM((CPT, CH), jnp.int32)]
    scratch += [pltpu.VMEM((CH,), jnp.int32) for _ in range(NSLOT)]
    scratch += [pltpu.VMEM((CH, L), jnp.float32) for _ in range(NSLOT)]
    scratch += [pltpu.VMEM_SHARED((NP, L), jnp.float32) for _ in range(NACC)]
    scratch += [
        pltpu.SemaphoreType.DMA,
        pltpu.SemaphoreType.DMA,
        pltpu.SemaphoreType.DMA,
    ]

    @functools.partial(
        pl.kernel,
        out_type=jax.ShapeDtypeStruct((NC, NACC, NP, L), jnp.float32),
        mesh=mesh,
        scratch_types=scratch,
        compiler_params=pltpu.CompilerParams(use_tc_tiling_on_sc=False),
    )
    def seg_kernel(z_hbm, src2d_hbm, dst_hbm, zero_hbm, sp_hbm, *scr):
        idx_s = scr[0]
        dbufs = scr[1:1 + NSLOT]
        rbufs = scr[1 + NSLOT:1 + 2 * NSLOT]
        accs = scr[1 + 2 * NSLOT:1 + 2 * NSLOT + NACC]
        semi, semg, sems = scr[1 + 2 * NSLOT + NACC:]
        c = lax.axis_index("c")
        s = lax.axis_index("s")
        wid = s * NC + c

        for p in range(NACC):
            pltpu.sync_copy(zero_hbm.at[pl.ds(s * RPT, RPT)],
                            accs[p].at[pl.ds(s * RPT, RPT)])
        pltpu.sync_copy(src2d_hbm.at[pl.ds(wid * CPT, CPT)], idx_s)
        plsc.subcore_barrier()

        def idxcp(j, slot):
            base = (wid * CPT + j) * CH
            return pltpu.async_copy(dst_hbm.at[pl.ds(base, CH)],
                                    dbufs[slot], semi)

        def gather(j, slot):
            return pltpu.async_copy(z_hbm.at[idx_s.at[j]], rbufs[slot], semg)

        ids, gds = {}, {}
        sprev = [None] * NACC
        for b in range(G):
            gds[b] = gather(b, b)
            ids[b] = idxcp(b, b)
        for g in range(NG):
            base = g * G
            for b in range(G):
                gds[base + b].wait()
            for p in range(NACC):
                if sprev[p] is not None:
                    sprev[p].wait()
                    sprev[p] = None
            if g + 1 < NG:
                for b in range(G):
                    j = (g + 1) * G + b
                    slot = ((g + 1) % 2) * G + b
                    gds[j] = gather(j, slot)
                    ids[j] = idxcp(j, slot)
            for b in range(G):
                ids[base + b].wait()
            for b in range(G):
                slot = (g % 2) * G + b
                p = b % NACC
                if sprev[p] is not None:
                    sprev[p].wait()
                sprev[p] = pltpu.async_copy(
                    rbufs[slot], accs[p].at[dbufs[slot]], sems, add=True)
        for p in range(NACC):
            if sprev[p] is not None:
                sprev[p].wait()

        plsc.subcore_barrier()
        for p in range(NACC):
            pltpu.sync_copy(accs[p].at[pl.ds(s * RPT, RPT)],
                            sp_hbm.at[c, p, pl.ds(s * RPT, RPT)])

    return seg_kernel


def _tc_b(x_p, W0_1, W1_1, degp, NP):
    def body(x_ref, w0_ref, w1_ref, degp_ref, z1_ref, xw0_ref, dinv_ref):
        deg = jnp.sum(degp_ref[...].reshape(NW, NP), axis=0)
        dinv = jnp.where(deg > 0.0, lax.rsqrt(deg), 0.0)
        xv = x_ref[...]
        y1 = jnp.dot(xv, w1_ref[...], preferred_element_type=jnp.float32)
        z1_ref[...] = y1 * dinv[:, None]
        xw0_ref[...] = jnp.dot(xv, w0_ref[...],
                               preferred_element_type=jnp.float32)
        dinv_ref[...] = dinv

    return pl.pallas_call(
        body,
        out_shape=(
            jax.ShapeDtypeStruct((NP, L), jnp.float32),
            jax.ShapeDtypeStruct((NP, L), jnp.float32),
            jax.ShapeDtypeStruct((NP,), jnp.float32),
        ),
    )(x_p, W0_1, W1_1, degp)


def _tc_d(xw0, s1p, dinv, b1, NP):
    def body(xw0_ref, s1p_ref, dinv_ref, b1_ref, h_ref, z2_ref):
        s1 = jnp.sum(s1p_ref[...], axis=(0, 1))
        dv = dinv_ref[...][:, None]
        h = jnp.maximum(xw0_ref[...] - dv * s1 + b1_ref[...], 0.0)
        h_ref[...] = h
        z2_ref[...] = dv * h

    return pl.pallas_call(
        body,
        out_shape=(
            jax.ShapeDtypeStruct((NP, L), jnp.float32),
            jax.ShapeDtypeStruct((NP, L), jnp.float32),
        ),
    )(xw0, s1p, dinv, b1.reshape(1, L))


def _tc_f(h, s2p, dinv, W0_2, W1_2, b2, NP, D_out):
    def body(h_ref, s2p_ref, dinv_ref, w0_ref, w1_ref, b2_ref, out_ref):
        dv = dinv_ref[...][:, None]
        t = -dv * jnp.sum(s2p_ref[...], axis=(0, 1))
        out_ref[...] = (
            jnp.dot(h_ref[...], w0_ref[...], preferred_element_type=jnp.float32)
            + jnp.dot(t, w1_ref[...], preferred_element_type=jnp.float32)
            + b2_ref[...]
        )

    return pl.pallas_call(
        body,
        out_shape=jax.ShapeDtypeStruct((NP, D_out), jnp.float32),
    )(h, s2p, dinv, W0_2, W1_2, b2.reshape(1, D_out))


def kernel(x, edge_index, W0_1, W1_1, b1, W0_2, W1_2, b2):
    N, _ = x.shape
    E = edge_index.shape[1]
    D_out = W0_2.shape[1]

    NP = ((N + 1 + 1279) // 1280) * 1280
    CPT = (-(-E // (NW * CH)) + 7) // 8 * 8
    EP = NW * CPT * CH

    src = edge_index[0].astype(jnp.int32)
    dst = edge_index[1].astype(jnp.int32)
    pad = jnp.full((EP - E,), N, jnp.int32)
    src_flat = jnp.concatenate([src, pad])
    dst_flat = jnp.concatenate([dst, pad])
    src2d = src_flat.reshape(EP // CH, CH)
    x_p = jnp.concatenate(
        [x, jnp.zeros((NP - N, x.shape[1]), jnp.float32)], axis=0)
    zeros_nl = jnp.zeros((NP, L), jnp.float32)

    degp = _make_sc_deg(NP, CPT)(src_flat)
    z1, xw0, dinv = _tc_b(x_p, W0_1, W1_1, degp, NP)
    seg = _make_sc_segsum(NP, CPT)
    s1p = seg(z1, src2d, dst_flat, zeros_nl)
    h, z2 = _tc_d(xw0, s1p, dinv, b1, NP)
    s2p = seg(z2, src2d, dst_flat, zeros_nl)
    out = _tc_f(h, s2p, dinv, W0_2, W1_2, b2, NP, D_out)
    return out[:N]


# speedup vs baseline: 1.9330x; 1.9330x over previous
import functools

import jax
import jax.numpy as jnp
from jax import lax
from jax.experimental import pallas as pl
from jax.experimental.pallas import tpu as pltpu
from jax.experimental.pallas import tpu_sc as plsc

NC = 2
NS = 16
NW = NC * NS
L = 16
CH = 128


def _make_sc_deg(NP, CPT):
    mesh = plsc.VectorSubcoreMesh(core_axis_name="c", subcore_axis_name="s")

    @functools.partial(
        pl.kernel,
        out_type=jax.ShapeDtypeStruct((NW * NP,), jnp.float32),
        mesh=mesh,
        scratch_types=[
            pltpu.VMEM((CPT * CH,), jnp.int32),
            pltpu.VMEM((NP,), jnp.float32),
        ],
        compiler_params=pltpu.CompilerParams(needs_layout_passes=False),
    )
    def deg_kernel(src_hbm, degp_hbm, idx_v, deg_v):
        c = lax.axis_index("c")
        s = lax.axis_index("s")
        wid = s * NC + c

        def zero_body(i, _):
            deg_v[pl.ds(i * L, L)] = jnp.zeros((L,), jnp.float32)
            return 0

        lax.fori_loop(0, NP // L, zero_body, 0)

        pltpu.sync_copy(src_hbm.at[pl.ds(wid * CPT * CH, CPT * CH)], idx_v)

        ones = jnp.ones((L,), jnp.float32)

        def body(j, _):
            for i in range(CH // L):
                iv = idx_v[pl.ds(j * CH + i * L, L)]
                plsc.addupdate_scatter(deg_v, [iv], ones)
            return 0

        lax.fori_loop(0, CPT, body, 0)
        pltpu.sync_copy(deg_v, degp_hbm.at[pl.ds(wid * NP, NP)])

    return deg_kernel


def _make_sc_segsum(NP, CPT):
    mesh = plsc.VectorSubcoreMesh(core_axis_name="c", subcore_axis_name="s")
    RPT = NP // NS
    G = 8
    NG = CPT // G
    NSLOT = 2 * G

    scratch = [pltpu.VMEM((CPT, CH), jnp.int32)]
    scratch += [pltpu.VMEM((CH,), jnp.int32) for _ in range(NSLOT)]
    scratch += [pltpu.VMEM((CH, L), jnp.float32) for _ in range(NSLOT)]
    scratch += [
        pltpu.VMEM_SHARED((NP, L), jnp.float32),
        pltpu.VMEM_SHARED((NP, L), jnp.float32),
        pltpu.SemaphoreType.DMA,
        pltpu.SemaphoreType.DMA,
    ]

    @functools.partial(
        pl.kernel,
        out_type=jax.ShapeDtypeStruct((NC, NP, L), jnp.float32),
        mesh=mesh,
        scratch_types=scratch,
        compiler_params=pltpu.CompilerParams(use_tc_tiling_on_sc=False),
    )
    def seg_kernel(z_hbm, src2d_hbm, dst_hbm, zero_hbm, sp_hbm, *scr):
        idx_s = scr[0]
        dbufs = scr[1:1 + NSLOT]
        rbufs = scr[1 + NSLOT:1 + 2 * NSLOT]
        acc, z_s, semi, semg = scr[1 + 2 * NSLOT:]
        c = lax.axis_index("c")
        s = lax.axis_index("s")
        wid = s * NC + c

        pltpu.sync_copy(z_hbm.at[pl.ds(s * RPT, RPT)],
                        z_s.at[pl.ds(s * RPT, RPT)])
        pltpu.sync_copy(zero_hbm.at[pl.ds(s * RPT, RPT)],
                        acc.at[pl.ds(s * RPT, RPT)])
        pltpu.sync_copy(src2d_hbm.at[pl.ds(wid * CPT, CPT)], idx_s)
        plsc.subcore_barrier()

        def idxcp(j, slot):
            base = (wid * CPT + j) * CH
            return pltpu.async_copy(dst_hbm.at[pl.ds(base, CH)],
                                    dbufs[slot], semi)

        def gather(j, slot):
            return pltpu.async_copy(z_s.at[idx_s.at[j]], rbufs[slot], semg)

        def scatter(slot):
            pltpu.sync_copy(rbufs[slot], acc.at[dbufs[slot]], add=True)

        ids, gds = {}, {}
        for b in range(G):
            gds[b] = gather(b, b)
            ids[b] = idxcp(b, b)
        for g in range(NG):
            base = g * G
            for b in range(G):
                gds[base + b].wait()
            if g + 1 < NG:
                for b in range(G):
                    j = (g + 1) * G + b
                    slot = ((g + 1) % 2) * G + b
                    gds[j] = gather(j, slot)
                    ids[j] = idxcp(j, slot)
            for b in range(G):
                ids[base + b].wait()
            for b in range(G):
                scatter((g % 2) * G + b)

        plsc.subcore_barrier()
        pltpu.sync_copy(acc.at[pl.ds(s * RPT, RPT)],
                        sp_hbm.at[c, pl.ds(s * RPT, RPT)])

    return seg_kernel


def _tc_b(x_p, W0_1, W1_1, degp, NP):
    def body(x_ref, w0_ref, w1_ref, degp_ref, z1_ref, xw0_ref, dinv_ref):
        deg = jnp.sum(degp_ref[...].reshape(NW, NP), axis=0)
        dinv = jnp.where(deg > 0.0, lax.rsqrt(deg), 0.0)
        xv = x_ref[...]
        y1 = jnp.dot(xv, w1_ref[...], preferred_element_type=jnp.float32)
        z1_ref[...] = y1 * dinv[:, None]
        xw0_ref[...] = jnp.dot(xv, w0_ref[...],
                               preferred_element_type=jnp.float32)
        dinv_ref[...] = dinv

    return pl.pallas_call(
        body,
        out_shape=(
            jax.ShapeDtypeStruct((NP, L), jnp.float32),
            jax.ShapeDtypeStruct((NP, L), jnp.float32),
            jax.ShapeDtypeStruct((NP,), jnp.float32),
        ),
    )(x_p, W0_1, W1_1, degp)


def _tc_d(xw0, s1p, dinv, b1, NP):
    def body(xw0_ref, s1p_ref, dinv_ref, b1_ref, h_ref, z2_ref):
        s1 = s1p_ref[0] + s1p_ref[1]
        dv = dinv_ref[...][:, None]
        h = jnp.maximum(xw0_ref[...] - dv * s1 + b1_ref[...], 0.0)
        h_ref[...] = h
        z2_ref[...] = dv * h

    return pl.pallas_call(
        body,
        out_shape=(
            jax.ShapeDtypeStruct((NP, L), jnp.float32),
            jax.ShapeDtypeStruct((NP, L), jnp.float32),
        ),
    )(xw0, s1p, dinv, b1.reshape(1, L))


def _tc_f(h, s2p, dinv, W0_2, W1_2, b2, NP, D_out):
    def body(h_ref, s2p_ref, dinv_ref, w0_ref, w1_ref, b2_ref, out_ref):
        dv = dinv_ref[...][:, None]
        t = -dv * (s2p_ref[0] + s2p_ref[1])
        out_ref[...] = (
            jnp.dot(h_ref[...], w0_ref[...], preferred_element_type=jnp.float32)
            + jnp.dot(t, w1_ref[...], preferred_element_type=jnp.float32)
            + b2_ref[...]
        )

    return pl.pallas_call(
        body,
        out_shape=jax.ShapeDtypeStruct((NP, D_out), jnp.float32),
    )(h, s2p, dinv, W0_2, W1_2, b2.reshape(1, D_out))


def kernel(x, edge_index, W0_1, W1_1, b1, W0_2, W1_2, b2):
    N, _ = x.shape
    E = edge_index.shape[1]
    D_out = W0_2.shape[1]

    NP = ((N + 1 + 1279) // 1280) * 1280
    CPT = (-(-E // (NW * CH)) + 7) // 8 * 8
    EP = NW * CPT * CH

    src = edge_index[0].astype(jnp.int32)
    dst = edge_index[1].astype(jnp.int32)
    pad = jnp.full((EP - E,), N, jnp.int32)
    src_flat = jnp.concatenate([src, pad])
    dst_flat = jnp.concatenate([dst, pad])
    src2d = src_flat.reshape(EP // CH, CH)
    x_p = jnp.concatenate(
        [x, jnp.zeros((NP - N, x.shape[1]), jnp.float32)], axis=0)
    zeros_nl = jnp.zeros((NP, L), jnp.float32)

    degp = _make_sc_deg(NP, CPT)(src_flat)
    z1, xw0, dinv = _tc_b(x_p, W0_1, W1_1, degp, NP)
    seg = _make_sc_segsum(NP, CPT)
    s1p = seg(z1, src2d, dst_flat, zeros_nl)
    h, z2 = _tc_d(xw0, s1p, dinv, b1, NP)
    s2p = seg(z2, src2d, dst_flat, zeros_nl)
    out = _tc_f(h, s2p, dinv, W0_2, W1_2, b2, NP, D_out)
    return out[:N]


# speedup vs baseline: 1.9378x; 1.0025x over previous
import functools

import jax
import jax.numpy as jnp
from jax import lax
from jax.experimental import pallas as pl
from jax.experimental.pallas import tpu as pltpu
from jax.experimental.pallas import tpu_sc as plsc

NC = 2
NS = 16
NW = NC * NS
L = 16
CH = 128


def _make_sc_deg(NP, CPT):
    mesh = plsc.VectorSubcoreMesh(core_axis_name="c", subcore_axis_name="s")

    @functools.partial(
        pl.kernel,
        out_type=jax.ShapeDtypeStruct((NW * NP,), jnp.float32),
        mesh=mesh,
        scratch_types=[
            pltpu.VMEM((CPT * CH,), jnp.int32),
            pltpu.VMEM((NP,), jnp.float32),
        ],
        compiler_params=pltpu.CompilerParams(needs_layout_passes=False),
    )
    def deg_kernel(src_hbm, degp_hbm, idx_v, deg_v):
        c = lax.axis_index("c")
        s = lax.axis_index("s")
        wid = s * NC + c

        def zero_body(i, _):
            deg_v[pl.ds(i * L, L)] = jnp.zeros((L,), jnp.float32)
            return 0

        lax.fori_loop(0, NP // L, zero_body, 0)

        pltpu.sync_copy(src_hbm.at[pl.ds(wid * CPT * CH, CPT * CH)], idx_v)

        ones = jnp.ones((L,), jnp.float32)

        UNR = 4

        def body(j, _):
            for i in range(UNR * CH // L):
                iv = idx_v[pl.ds(j * (UNR * CH) + i * L, L)]
                plsc.addupdate_scatter(deg_v, [iv], ones)
            return 0

        lax.fori_loop(0, CPT // UNR, body, 0)
        for jj in range(CPT // UNR * UNR, CPT):
            for i in range(CH // L):
                iv = idx_v[pl.ds(jj * CH + i * L, L)]
                plsc.addupdate_scatter(deg_v, [iv], ones)
        pltpu.sync_copy(deg_v, degp_hbm.at[pl.ds(wid * NP, NP)])

    return deg_kernel


def _make_sc_segsum(NP, CPT):
    mesh = plsc.VectorSubcoreMesh(core_axis_name="c", subcore_axis_name="s")
    RPT = NP // NS
    G = 8
    NG = CPT // G
    NSLOT = 2 * G

    scratch = [pltpu.VMEM((CPT, CH), jnp.int32)]
    scratch += [pltpu.VMEM((CH,), jnp.int32) for _ in range(NSLOT)]
    scratch += [pltpu.VMEM((CH, L), jnp.float32) for _ in range(NSLOT)]
    scratch += [
        pltpu.VMEM_SHARED((NP, L), jnp.float32),
        pltpu.VMEM_SHARED((NP, L), jnp.float32),
        pltpu.SemaphoreType.DMA,
        pltpu.SemaphoreType.DMA,
    ]

    @functools.partial(
        pl.kernel,
        out_type=jax.ShapeDtypeStruct((NC, NP, L), jnp.float32),
        mesh=mesh,
        scratch_types=scratch,
        compiler_params=pltpu.CompilerParams(use_tc_tiling_on_sc=False),
    )
    def seg_kernel(z_hbm, src2d_hbm, dst_hbm, zero_hbm, sp_hbm, *scr):
        idx_s = scr[0]
        dbufs = scr[1:1 + NSLOT]
        rbufs = scr[1 + NSLOT:1 + 2 * NSLOT]
        acc, z_s, semi, semg = scr[1 + 2 * NSLOT:]
        c = lax.axis_index("c")
        s = lax.axis_index("s")
        wid = s * NC + c

        pltpu.sync_copy(z_hbm.at[pl.ds(s * RPT, RPT)],
                        z_s.at[pl.ds(s * RPT, RPT)])
        pltpu.sync_copy(zero_hbm.at[pl.ds(s * RPT, RPT)],
                        acc.at[pl.ds(s * RPT, RPT)])
        pltpu.sync_copy(src2d_hbm.at[pl.ds(wid * CPT, CPT)], idx_s)
        plsc.subcore_barrier()

        def idxcp(j, slot):
            base = (wid * CPT + j) * CH
            return pltpu.async_copy(dst_hbm.at[pl.ds(base, CH)],
                                    dbufs[slot], semi)

        def gather(j, slot):
            return pltpu.async_copy(z_s.at[idx_s.at[j]], rbufs[slot], semg)

        def scatter(slot):
            pltpu.sync_copy(rbufs[slot], acc.at[dbufs[slot]], add=True)

        ids, gds = {}, {}
        for b in range(G):
            gds[b] = gather(b, b)
            ids[b] = idxcp(b, b)
        for g in range(NG):
            base = g * G
            for b in range(G):
                gds[base + b].wait()
            if g + 1 < NG:
                for b in range(G):
                    j = (g + 1) * G + b
                    slot = ((g + 1) % 2) * G + b
                    gds[j] = gather(j, slot)
                    ids[j] = idxcp(j, slot)
            for b in range(G):
                ids[base + b].wait()
            for b in range(G):
                scatter((g % 2) * G + b)

        plsc.subcore_barrier()
        pltpu.sync_copy(acc.at[pl.ds(s * RPT, RPT)],
                        sp_hbm.at[c, pl.ds(s * RPT, RPT)])

    return seg_kernel


def _tc_b1(x_p, W0_1, W1_1, NP):
    def body(x_ref, w0_ref, w1_ref, y1_ref, xw0_ref):
        xv = x_ref[...]
        y1_ref[...] = jnp.dot(xv, w1_ref[...],
                              preferred_element_type=jnp.float32)
        xw0_ref[...] = jnp.dot(xv, w0_ref[...],
                               preferred_element_type=jnp.float32)

    return pl.pallas_call(
        body,
        out_shape=(
            jax.ShapeDtypeStruct((NP, L), jnp.float32),
            jax.ShapeDtypeStruct((NP, L), jnp.float32),
        ),
    )(x_p, W0_1, W1_1)


def _tc_b2(y1, degp, NP):
    def body(y1_ref, degp_ref, z1_ref, dinv_ref):
        deg = jnp.sum(degp_ref[...].reshape(NW, NP), axis=0)
        dinv = jnp.where(deg > 0.0, lax.rsqrt(deg), 0.0)
        z1_ref[...] = y1_ref[...] * dinv[:, None]
        dinv_ref[...] = dinv

    return pl.pallas_call(
        body,
        out_shape=(
            jax.ShapeDtypeStruct((NP, L), jnp.float32),
            jax.ShapeDtypeStruct((NP,), jnp.float32),
        ),
    )(y1, degp)


def _tc_d(xw0, s1p, dinv, b1, NP):
    def body(xw0_ref, s1p_ref, dinv_ref, b1_ref, h_ref, z2_ref):
        s1 = s1p_ref[0] + s1p_ref[1]
        dv = dinv_ref[...][:, None]
        h = jnp.maximum(xw0_ref[...] - dv * s1 + b1_ref[...], 0.0)
        h_ref[...] = h
        z2_ref[...] = dv * h

    return pl.pallas_call(
        body,
        out_shape=(
            jax.ShapeDtypeStruct((NP, L), jnp.float32),
            jax.ShapeDtypeStruct((NP, L), jnp.float32),
        ),
    )(xw0, s1p, dinv, b1.reshape(1, L))


def _tc_f1(h, W0_2, b2, NP, D_out):
    def body(h_ref, w0_ref, b2_ref, hw_ref):
        hw_ref[...] = jnp.dot(h_ref[...], w0_ref[...],
                              preferred_element_type=jnp.float32) + b2_ref[...]

    return pl.pallas_call(
        body,
        out_shape=jax.ShapeDtypeStruct((NP, D_out), jnp.float32),
    )(h, W0_2, b2.reshape(1, D_out))


def _tc_f2(hw, s2p, dinv, W1_2, NP, D_out):
    def body(hw_ref, s2p_ref, dinv_ref, w1_ref, out_ref):
        t = -dinv_ref[...][:, None] * (s2p_ref[0] + s2p_ref[1])
        out_ref[...] = hw_ref[...] + jnp.dot(
            t, w1_ref[...], preferred_element_type=jnp.float32)

    return pl.pallas_call(
        body,
        out_shape=jax.ShapeDtypeStruct((NP, D_out), jnp.float32),
    )(hw, s2p, dinv, W1_2)


def kernel(x, edge_index, W0_1, W1_1, b1, W0_2, W1_2, b2):
    N, _ = x.shape
    E = edge_index.shape[1]
    D_out = W0_2.shape[1]

    NP = ((N + 1 + 1279) // 1280) * 1280
    CPT = (-(-E // (NW * CH)) + 7) // 8 * 8
    EP = NW * CPT * CH

    src = edge_index[0].astype(jnp.int32)
    dst = edge_index[1].astype(jnp.int32)
    pad = jnp.full((EP - E,), N, jnp.int32)
    src_flat = jnp.concatenate([src, pad])
    dst_flat = jnp.concatenate([dst, pad])
    src2d = src_flat.reshape(EP // CH, CH)
    x_p = jnp.concatenate(
        [x, jnp.zeros((NP - N, x.shape[1]), jnp.float32)], axis=0)
    zeros_nl = jnp.zeros((NP, L), jnp.float32)

    degp = _make_sc_deg(NP, CPT)(src_flat)
    y1, xw0 = _tc_b1(x_p, W0_1, W1_1, NP)
    z1, dinv = _tc_b2(y1, degp, NP)
    seg = _make_sc_segsum(NP, CPT)
    s1p = seg(z1, src2d, dst_flat, zeros_nl)
    h, z2 = _tc_d(xw0, s1p, dinv, b1, NP)
    hw = _tc_f1(h, W0_2, b2, NP, D_out)
    s2p = seg(z2, src2d, dst_flat, zeros_nl)
    out = _tc_f2(hw, s2p, dinv, W1_2, NP, D_out)
    return out[:N]


# speedup vs baseline: 1.9940x; 1.0290x over previous
import functools

import jax
import jax.numpy as jnp
from jax import lax
from jax.experimental import pallas as pl
from jax.experimental.pallas import tpu as pltpu
from jax.experimental.pallas import tpu_sc as plsc

NC = 2
NS = 16
NW = NC * NS
L = 16
CH = 128


def _make_sc_deg(NP, CPT):
    mesh = plsc.VectorSubcoreMesh(core_axis_name="c", subcore_axis_name="s")

    @functools.partial(
        pl.kernel,
        out_type=jax.ShapeDtypeStruct((NW * NP,), jnp.float32),
        mesh=mesh,
        scratch_types=[
            pltpu.VMEM((CPT * CH,), jnp.int32),
            pltpu.VMEM((NP,), jnp.float32),
        ],
        compiler_params=pltpu.CompilerParams(needs_layout_passes=False),
    )
    def deg_kernel(src_hbm, degp_hbm, idx_v, deg_v):
        c = lax.axis_index("c")
        s = lax.axis_index("s")
        wid = s * NC + c

        def zero_body(i, _):
            deg_v[pl.ds(i * L, L)] = jnp.zeros((L,), jnp.float32)
            return 0

        lax.fori_loop(0, NP // L, zero_body, 0)

        pltpu.sync_copy(src_hbm.at[pl.ds(wid * CPT * CH, CPT * CH)], idx_v)

        ones = jnp.ones((L,), jnp.float32)

        UNR = 4

        def body(j, _):
            for i in range(UNR * CH // L):
                iv = idx_v[pl.ds(j * (UNR * CH) + i * L, L)]
                plsc.addupdate_scatter(deg_v, [iv], ones)
            return 0

        lax.fori_loop(0, CPT // UNR, body, 0)
        for jj in range(CPT // UNR * UNR, CPT):
            for i in range(CH // L):
                iv = idx_v[pl.ds(jj * CH + i * L, L)]
                plsc.addupdate_scatter(deg_v, [iv], ones)
        pltpu.sync_copy(deg_v, degp_hbm.at[pl.ds(wid * NP, NP)])

    return deg_kernel


def _make_sc_segsum(NP, EP):
    mesh = plsc.VectorSubcoreMesh(core_axis_name="c", subcore_axis_name="s")
    RPT = NP // NS
    BASE = EP // CH // NW
    G = 8
    NSLOT = 2 * G

    scratch = [pltpu.VMEM((BASE, CH), jnp.int32)]
    scratch += [pltpu.VMEM((CH,), jnp.int32) for _ in range(NSLOT)]
    scratch += [pltpu.VMEM((CH, L), jnp.float32) for _ in range(NSLOT)]
    scratch += [
        pltpu.VMEM_SHARED((NP, L), jnp.float32),
        pltpu.VMEM_SHARED((NP, L), jnp.float32),
        pltpu.SemaphoreType.DMA,
        pltpu.SemaphoreType.DMA,
    ]

    @functools.partial(
        pl.kernel,
        out_type=jax.ShapeDtypeStruct((NC, NP, L), jnp.float32),
        mesh=mesh,
        scratch_types=scratch,
        compiler_params=pltpu.CompilerParams(use_tc_tiling_on_sc=False),
    )
    def seg_kernel(z_hbm, src2d_hbm, dst_hbm, zero_hbm, sp_hbm, *scr):
        idx_s = scr[0]
        dbufs = scr[1:1 + NSLOT]
        rbufs = scr[1 + NSLOT:1 + 2 * NSLOT]
        acc, z_s, semi, semg = scr[1 + 2 * NSLOT:]
        c = lax.axis_index("c")
        s = lax.axis_index("s")
        wid = s * NC + c
        cbase = wid * BASE

        pltpu.sync_copy(z_hbm.at[pl.ds(s * RPT, RPT)],
                        z_s.at[pl.ds(s * RPT, RPT)])
        pltpu.sync_copy(zero_hbm.at[pl.ds(s * RPT, RPT)],
                        acc.at[pl.ds(s * RPT, RPT)])
        pltpu.sync_copy(src2d_hbm.at[pl.ds(cbase, BASE)], idx_s)
        plsc.subcore_barrier()

        def idxcp(j, slot):
            return pltpu.async_copy(
                dst_hbm.at[pl.ds((cbase + j) * CH, CH)], dbufs[slot], semi)

        def gather(j, slot):
            return pltpu.async_copy(z_s.at[idx_s.at[j]], rbufs[slot], semg)

        def scatter(slot):
            pltpu.sync_copy(rbufs[slot], acc.at[dbufs[slot]], add=True)

        groups = [list(range(g, min(g + G, BASE))) for g in range(0, BASE, G)]
        ids, gds = {}, {}
        for b, j in enumerate(groups[0]):
            gds[j] = gather(j, b)
            ids[j] = idxcp(j, b)
        for gi, grp in enumerate(groups):
            for j in grp:
                gds[j].wait()
            if gi + 1 < len(groups):
                for b, j in enumerate(groups[gi + 1]):
                    slot = ((gi + 1) % 2) * G + b
                    gds[j] = gather(j, slot)
                    ids[j] = idxcp(j, slot)
            for j in grp:
                ids[j].wait()
            for b, j in enumerate(grp):
                scatter((gi % 2) * G + b)

        plsc.subcore_barrier()
        pltpu.sync_copy(acc.at[pl.ds(s * RPT, RPT)],
                        sp_hbm.at[c, pl.ds(s * RPT, RPT)])

    return seg_kernel


def _tc_b1(x, W0_1, W1_1, NP):
    N = x.shape[0]

    def body(x_ref, w0_ref, w1_ref, y1_ref, xw0_ref):
        xv = x_ref[...]
        zpad = jnp.zeros((NP - N, L), jnp.float32)
        y1_ref[...] = jnp.concatenate(
            [jnp.dot(xv, w1_ref[...], preferred_element_type=jnp.float32),
             zpad], axis=0)
        xw0_ref[...] = jnp.concatenate(
            [jnp.dot(xv, w0_ref[...], preferred_element_type=jnp.float32),
             zpad], axis=0)

    return pl.pallas_call(
        body,
        out_shape=(
            jax.ShapeDtypeStruct((NP, L), jnp.float32),
            jax.ShapeDtypeStruct((NP, L), jnp.float32),
        ),
    )(x, W0_1, W1_1)


def _tc_b2(y1, degp, NP):
    def body(y1_ref, degp_ref, z1_ref, dinv_ref):
        deg = jnp.sum(degp_ref[...].reshape(NW, NP), axis=0)
        dinv = jnp.where(deg > 0.0, lax.rsqrt(deg), 0.0)
        z1_ref[...] = y1_ref[...] * dinv[:, None]
        dinv_ref[...] = dinv

    return pl.pallas_call(
        body,
        out_shape=(
            jax.ShapeDtypeStruct((NP, L), jnp.float32),
            jax.ShapeDtypeStruct((NP,), jnp.float32),
        ),
    )(y1, degp)


def _tc_d(xw0, s1p, dinv, b1, NP):
    def body(xw0_ref, s1p_ref, dinv_ref, b1_ref, h_ref, z2_ref):
        s1 = s1p_ref[0] + s1p_ref[1]
        dv = dinv_ref[...][:, None]
        h = jnp.maximum(xw0_ref[...] - dv * s1 + b1_ref[...], 0.0)
        h_ref[...] = h
        z2_ref[...] = dv * h

    return pl.pallas_call(
        body,
        out_shape=(
            jax.ShapeDtypeStruct((NP, L), jnp.float32),
            jax.ShapeDtypeStruct((NP, L), jnp.float32),
        ),
    )(xw0, s1p, dinv, b1.reshape(1, L))


def _tc_f1(h, W0_2, b2, NP, D_out):
    def body(h_ref, w0_ref, b2_ref, hw_ref):
        hw_ref[...] = jnp.dot(h_ref[...], w0_ref[...],
                              preferred_element_type=jnp.float32) + b2_ref[...]

    return pl.pallas_call(
        body,
        out_shape=jax.ShapeDtypeStruct((NP, D_out), jnp.float32),
    )(h, W0_2, b2.reshape(1, D_out))


def _tc_f2(hw, s2p, dinv, W1_2, N, D_out):
    def body(hw_ref, s2p_ref, dinv_ref, w1_ref, out_ref):
        t = -dinv_ref[...][:, None] * (s2p_ref[0] + s2p_ref[1])
        full = hw_ref[...] + jnp.dot(
            t, w1_ref[...], preferred_element_type=jnp.float32)
        out_ref[...] = full[:N]

    return pl.pallas_call(
        body,
        out_shape=jax.ShapeDtypeStruct((N, D_out), jnp.float32),
    )(hw, s2p, dinv, W1_2)


def kernel(x, edge_index, W0_1, W1_1, b1, W0_2, W1_2, b2):
    N, _ = x.shape
    E = edge_index.shape[1]
    D_out = W0_2.shape[1]

    NP = ((N + 1 + 1279) // 1280) * 1280

    BASE = ((-(-E // (NW * CH)) + 7) // 8) * 8
    EP = NW * BASE * CH
    src = jnp.pad(edge_index[0].astype(jnp.int32), (0, EP - E),
                  constant_values=N)
    dst = jnp.pad(edge_index[1].astype(jnp.int32), (0, EP - E),
                  constant_values=N)
    src2d = src.reshape(EP // CH, CH)
    zeros_nl = jnp.zeros((NP, L), jnp.float32)

    degp = _make_sc_deg(NP, BASE)(src)
    y1, xw0 = _tc_b1(x, W0_1, W1_1, NP)
    z1, dinv = _tc_b2(y1, degp, NP)
    seg = _make_sc_segsum(NP, EP)
    s1p = seg(z1, src2d, dst, zeros_nl)
    h, z2 = _tc_d(xw0, s1p, dinv, b1, NP)
    hw = _tc_f1(h, W0_2, b2, NP, D_out)
    s2p = seg(z2, src2d, dst, zeros_nl)
    out = _tc_f2(hw, s2p, dinv, W1_2, N, D_out)
    return out
